# baseline (speedup 1.0000x reference)
import functools

import jax
import jax.numpy as jnp
from jax import lax
from jax.experimental import pallas as pl
from jax.experimental.pallas import tpu as pltpu
from jax.experimental.pallas import tpu_sc as plsc

D_MODEL = 64
CHUNK = 128


@functools.cache
def _make_lookup(n_idx: int, d: int):
    info = plsc.get_sparse_core_info()
    nw = info.num_cores * info.num_subcores
    assert n_idx % (nw * CHUNK) == 0
    n_chunks = n_idx // (nw * CHUNK)
    mesh = plsc.VectorSubcoreMesh(core_axis_name="c", subcore_axis_name="s")

    @functools.partial(
        pl.kernel,
        mesh=mesh,
        out_type=jax.ShapeDtypeStruct((n_idx, d), jnp.float32),
        scratch_types=[
            pltpu.VMEM((n_chunks, CHUNK), jnp.int32),
            pltpu.VMEM((CHUNK, d), jnp.float32),
            pltpu.SemaphoreType.DMA,
        ],
        compiler_params=pltpu.CompilerParams(use_tc_tiling_on_sc=False),
    )
    def lookup(idx_hbm, table_hbm, out_hbm, idx_v, rows_v, sem):
        wid = lax.axis_index("s") * info.num_cores + lax.axis_index("c")
        pltpu.sync_copy(idx_hbm.at[wid], idx_v)

        def body(j, carry):
            pltpu.async_copy(table_hbm.at[idx_v.at[j]], rows_v, sem).wait()
            base = (wid * n_chunks + j) * CHUNK
            pltpu.sync_copy(rows_v, out_hbm.at[pl.ds(base, CHUNK)])
            return carry

        lax.fori_loop(0, n_chunks, body, 0)

    return lookup, nw, n_chunks


def kernel(x, embedding_weight):
    b, l = x.shape
    n_idx = b * l
    lookup, nw, n_chunks = _make_lookup(n_idx, D_MODEL)
    idx3 = x.reshape(nw, n_chunks, CHUNK).astype(jnp.int32)
    out = lookup(idx3, embedding_weight)
    return out.reshape(b, l, D_MODEL)


# speedup vs baseline: 1.0388x; 1.0388x over previous
import functools

import jax
import jax.numpy as jnp
from jax import lax
from jax.experimental import pallas as pl
from jax.experimental.pallas import tpu as pltpu
from jax.experimental.pallas import tpu_sc as plsc

D_MODEL = 64
CHUNK = 128


@functools.cache
def _make_lookup(n_idx: int, d: int):
    info = plsc.get_sparse_core_info()
    nw = info.num_cores * info.num_subcores
    assert n_idx % (nw * CHUNK) == 0
    n_chunks = n_idx // (nw * CHUNK)
    assert n_chunks % 2 == 0 and n_chunks >= 4
    mesh = plsc.VectorSubcoreMesh(core_axis_name="c", subcore_axis_name="s")

    @functools.partial(
        pl.kernel,
        mesh=mesh,
        out_type=jax.ShapeDtypeStruct((n_idx, d), jnp.float32),
        scratch_types=[
            pltpu.VMEM((n_chunks, CHUNK), jnp.int32),
            pltpu.VMEM((2, CHUNK, d), jnp.float32),
            pltpu.SemaphoreType.DMA,
            pltpu.SemaphoreType.DMA,
        ],
        compiler_params=pltpu.CompilerParams(use_tc_tiling_on_sc=False),
    )
    def lookup(idx_hbm, table_hbm, out_hbm, idx_v, rows_v, gsem, osem):
        wid = lax.axis_index("s") * info.num_cores + lax.axis_index("c")
        pltpu.sync_copy(idx_hbm.at[wid], idx_v)

        def gather_copy(j, b):
            return pltpu.make_async_copy(
                table_hbm.at[idx_v.at[j]], rows_v.at[b], gsem
            )

        def write_copy(j, b):
            base = (wid * n_chunks + j) * CHUNK
            return pltpu.make_async_copy(
                rows_v.at[b], out_hbm.at[pl.ds(base, CHUNK)], osem
            )

        gather_copy(0, 0).start()

        def body(j2, carry):
            for b in (0, 1):
                j = j2 * 2 + b
                gather_copy(j, b).wait()

                @pl.when(j > 0)
                def _():
                    write_copy(j - 1, 1 - b).wait()

                @pl.when(j < n_chunks - 1)
                def _():
                    gather_copy(j + 1, 1 - b).start()

                write_copy(j, b).start()
            return carry

        lax.fori_loop(0, n_chunks // 2, body, 0)
        write_copy(n_chunks - 1, 1).wait()

    return lookup, nw, n_chunks


def kernel(x, embedding_weight):
    b, l = x.shape
    n_idx = b * l
    lookup, nw, n_chunks = _make_lookup(n_idx, D_MODEL)
    idx3 = x.reshape(nw, n_chunks, CHUNK).astype(jnp.int32)
    out = lookup(idx3, embedding_weight)
    return out.reshape(b, l, D_MODEL)


# speedup vs baseline: 1.1132x; 1.0716x over previous
import functools

import jax
import jax.numpy as jnp
from jax import lax
from jax.experimental import pallas as pl
from jax.experimental.pallas import tpu as pltpu
from jax.experimental.pallas import tpu_sc as plsc

D_MODEL = 64
CHUNK = 128
K = 4


@functools.cache
def _make_lookup(n_idx: int, d: int):
    info = plsc.get_sparse_core_info()
    nw = info.num_cores * info.num_subcores
    assert n_idx % (nw * CHUNK) == 0
    n_chunks = n_idx // (nw * CHUNK)
    n_groups = n_chunks // K
    assert n_chunks % K == 0 and n_groups % 2 == 0 and n_groups >= 4
    mesh = plsc.VectorSubcoreMesh(core_axis_name="c", subcore_axis_name="s")

    @functools.partial(
        pl.kernel,
        mesh=mesh,
        out_type=jax.ShapeDtypeStruct((n_idx, d), jnp.float32),
        scratch_types=[
            pltpu.VMEM((n_chunks, CHUNK), jnp.int32),
            pltpu.VMEM((2, K, CHUNK, d), jnp.float32),
            pltpu.SemaphoreType.DMA,
            pltpu.SemaphoreType.DMA,
            pltpu.SemaphoreType.DMA,
            pltpu.SemaphoreType.DMA,
        ],
        compiler_params=pltpu.CompilerParams(use_tc_tiling_on_sc=False),
    )
    def lookup(idx_hbm, table_hbm, out_hbm, idx_v, rows_v, ga, gb, oa, ob):
        wid = lax.axis_index("s") * info.num_cores + lax.axis_index("c")
        gsem = (ga, gb)
        osem = (oa, ob)
        pltpu.sync_copy(idx_hbm.at[wid], idx_v)

        def gather(t, p, k):
            return pltpu.make_async_copy(
                table_hbm.at[idx_v.at[t * K + k]], rows_v.at[p, k], gsem[p]
            )

        def write(t, p, k):
            base = (wid * n_chunks + t * K + k) * CHUNK
            return pltpu.make_async_copy(
                rows_v.at[p, k], out_hbm.at[pl.ds(base, CHUNK)], osem[p]
            )

        def fire_gathers(t, p):
            for k in range(K):
                gather(t, p, k).start()

        def drain_gathers(t, p):
            for k in range(K):
                gather(t, p, k).wait()

        def fire_writes(t, p):
            for k in range(K):
                write(t, p, k).start()

        def drain_writes(t, p):
            for k in range(K):
                write(t, p, k).wait()

        fire_gathers(0, 0)
        drain_gathers(0, 0)
        fire_writes(0, 0)
        fire_gathers(1, 1)

        def body(i, carry):
            for p in (1, 0):
                t = 2 * i + 1 + (1 - p)
                drain_gathers(t, p)
                fire_writes(t, p)
                drain_writes(t - 1, 1 - p)
                fire_gathers(t + 1, 1 - p)
            return carry

        lax.fori_loop(0, (n_groups - 2) // 2, body, 0)

        t_last = n_groups - 1
        drain_gathers(t_last, 1)
        fire_writes(t_last, 1)
        drain_writes(t_last - 1, 0)
        drain_writes(t_last, 1)

    return lookup, nw, n_chunks


def kernel(x, embedding_weight):
    b, l = x.shape
    n_idx = b * l
    lookup, nw, n_chunks = _make_lookup(n_idx, D_MODEL)
    idx3 = x.reshape(nw, n_chunks, CHUNK).astype(jnp.int32)
    out = lookup(idx3, embedding_weight)
    return out.reshape(b, l, D_MODEL)


# speedup vs baseline: 1.1142x; 1.0008x over previous
import functools

import jax
import jax.numpy as jnp
from jax import lax
from jax.experimental import pallas as pl
from jax.experimental.pallas import tpu as pltpu
from jax.experimental.pallas import tpu_sc as plsc

D_MODEL = 64
CHUNK = 128
K = 5


@functools.cache
def _make_lookup(n_idx: int, d: int):
    info = plsc.get_sparse_core_info()
    nw = info.num_cores * info.num_subcores
    assert n_idx % (nw * CHUNK) == 0
    n_chunks = n_idx // (nw * CHUNK)
    n_groups = n_chunks // K
    assert n_chunks % K == 0 and n_groups % 2 == 0 and n_groups >= 4
    mesh = plsc.VectorSubcoreMesh(core_axis_name="c", subcore_axis_name="s")

    @functools.partial(
        pl.kernel,
        mesh=mesh,
        out_type=jax.ShapeDtypeStruct((n_idx, d), jnp.float32),
        scratch_types=[
            pltpu.VMEM((n_chunks, CHUNK), jnp.int32),
            pltpu.VMEM((2, K, CHUNK, d), jnp.float32),
            pltpu.SemaphoreType.DMA,
            pltpu.SemaphoreType.DMA,
            pltpu.SemaphoreType.DMA,
            pltpu.SemaphoreType.DMA,
        ],
        compiler_params=pltpu.CompilerParams(use_tc_tiling_on_sc=False),
    )
    def lookup(idx_hbm, table_hbm, out_hbm, idx_v, rows_v, ga, gb, oa, ob):
        wid = lax.axis_index("s") * info.num_cores + lax.axis_index("c")
        gsem = (ga, gb)
        osem = (oa, ob)
        pltpu.sync_copy(idx_hbm.at[wid], idx_v)

        def gather(t, p, k):
            return pltpu.make_async_copy(
                table_hbm.at[idx_v.at[t * K + k]], rows_v.at[p, k], gsem[p]
            )

        def write(t, p, k):
            base = (wid * n_chunks + t * K + k) * CHUNK
            return pltpu.make_async_copy(
                rows_v.at[p, k], out_hbm.at[pl.ds(base, CHUNK)], osem[p]
            )

        def fire_gathers(t, p):
            for k in range(K):
                gather(t, p, k).start()

        def drain_gathers(t, p):
            for k in range(K):
                gather(t, p, k).wait()

        def fire_writes(t, p):
            for k in range(K):
                write(t, p, k).start()

        def drain_writes(t, p):
            for k in range(K):
                write(t, p, k).wait()

        fire_gathers(0, 0)
        drain_gathers(0, 0)
        fire_writes(0, 0)
        fire_gathers(1, 1)

        def body(i, carry):
            for p in (1, 0):
                t = 2 * i + 1 + (1 - p)
                drain_gathers(t, p)
                fire_writes(t, p)
                drain_writes(t - 1, 1 - p)
                fire_gathers(t + 1, 1 - p)
            return carry

        lax.fori_loop(0, (n_groups - 2) // 2, body, 0)

        t_last = n_groups - 1
        drain_gathers(t_last, 1)
        fire_writes(t_last, 1)
        drain_writes(t_last - 1, 0)
        drain_writes(t_last, 1)

    return lookup, nw, n_chunks


def kernel(x, embedding_weight):
    b, l = x.shape
    n_idx = b * l
    lookup, nw, n_chunks = _make_lookup(n_idx, D_MODEL)
    idx3 = x.reshape(nw, n_chunks, CHUNK).astype(jnp.int32)
    out = lookup(idx3, embedding_weight)
    return out.reshape(b, l, D_MODEL)
